# baseline (speedup 1.0000x reference)
import functools

import jax
import jax.numpy as jnp
from jax.experimental import pallas as pl
from jax.experimental.pallas import tpu as pltpu

_N_E = 8192
_E_DIM = 32
_BETA = 0.25
_N_TOK = 8192
_BLK = 256
_NBLK = _N_TOK // _BLK


def _vq_body(z_ref, w_ref, w2_ref, zsq_ref, wsq_ref, iota_ref, ones_ref,
             oh_ref, zq_ref, idx_ref, loss_ref, perp_ref,
             counts_ref, lsum_ref):
    i = pl.program_id(0)
    zb = z_ref[...]
    w = w_ref[...]
    zsq = zsq_ref[...]
    wsq = wsq_ref[...]

    mm2 = jax.lax.dot_general(
        zb, w2_ref[...], dimension_numbers=(((1,), (1,)), ((), ())),
        preferred_element_type=jnp.float32)
    d = zsq + wsq - mm2

    half = _N_E // 2
    iota = iota_ref[...]
    d_a, d_b = d[:, :half], d[:, half:]
    i_a, i_b = iota[:, :half], iota[:, half:]
    v_a = jnp.min(d_a, axis=1, keepdims=True)
    j_a = jnp.min(jnp.where(d_a == v_a, i_a, _N_E), axis=1, keepdims=True)
    v_b = jnp.min(d_b, axis=1, keepdims=True)
    j_b = jnp.min(jnp.where(d_b == v_b, i_b, _N_E), axis=1, keepdims=True)
    v_a16 = v_a.astype(jnp.bfloat16).astype(jnp.float32)
    keep_a = (v_a16 < v_b) | ((v_a16 == v_b) & (j_a < j_b))
    idx = jnp.where(keep_a, j_a, j_b)

    oh = jnp.where(iota == idx, 1.0, 0.0).astype(jnp.float32)
    oh_ref[...] = oh
    idx_ref[...] = idx

    zq = jax.lax.dot_general(
        oh, w, dimension_numbers=(((1,), (0,)), ((), ())),
        precision=jax.lax.Precision.HIGHEST,
        preferred_element_type=jnp.float32)
    zq_ref[...] = zq

    diff = zq - zb
    part_loss = jnp.sum(diff * diff)
    part_counts = jax.lax.dot_general(
        ones_ref[...], oh, dimension_numbers=(((1,), (0,)), ((), ())),
        precision=jax.lax.Precision.HIGHEST,
        preferred_element_type=jnp.float32)

    @pl.when(i == 0)
    def _init():
        lsum_ref[0, 0] = part_loss
        counts_ref[...] = part_counts

    @pl.when(i > 0)
    def _acc():
        lsum_ref[0, 0] += part_loss
        counts_ref[...] += part_counts

    @pl.when(i == _NBLK - 1)
    def _fini():
        mse = lsum_ref[0, 0] / (_N_TOK * _E_DIM)
        loss_ref[...] = jnp.reshape(_BETA * mse + mse, (1, 1))
        e_mean = counts_ref[...] / _N_TOK
        ent = jnp.sum(e_mean * jnp.log(e_mean + 1e-10), axis=1, keepdims=True)
        perp_ref[...] = jnp.exp(-ent)


@functools.partial(jax.jit, static_argnames=())
def kernel(z, weight):
    zsq = jnp.sum(z ** 2, axis=1, keepdims=True)
    wsq = jnp.sum(weight ** 2, axis=1)[None, :]
    w2 = weight * 2.0
    col_iota = jax.lax.broadcasted_iota(jnp.int32, (1, _N_E), 1)
    ones_row = jnp.ones((1, _BLK), jnp.float32)

    oh, zq, idx, loss, perp = pl.pallas_call(
        _vq_body,
        grid=(_NBLK,),
        in_specs=[
            pl.BlockSpec((_BLK, _E_DIM), lambda i: (i, 0)),
            pl.BlockSpec((_N_E, _E_DIM), lambda i: (0, 0)),
            pl.BlockSpec((_N_E, _E_DIM), lambda i: (0, 0)),
            pl.BlockSpec((_BLK, 1), lambda i: (i, 0)),
            pl.BlockSpec((1, _N_E), lambda i: (0, 0)),
            pl.BlockSpec((1, _N_E), lambda i: (0, 0)),
            pl.BlockSpec((1, _BLK), lambda i: (0, 0)),
        ],
        out_specs=[
            pl.BlockSpec((_BLK, _N_E), lambda i: (i, 0)),
            pl.BlockSpec((_BLK, _E_DIM), lambda i: (i, 0)),
            pl.BlockSpec((_BLK, 1), lambda i: (i, 0)),
            pl.BlockSpec((1, 1), lambda i: (0, 0)),
            pl.BlockSpec((1, 1), lambda i: (0, 0)),
        ],
        out_shape=[
            jax.ShapeDtypeStruct((_N_TOK, _N_E), jnp.float32),
            jax.ShapeDtypeStruct((_N_TOK, _E_DIM), jnp.float32),
            jax.ShapeDtypeStruct((_N_TOK, 1), jnp.int32),
            jax.ShapeDtypeStruct((1, 1), jnp.float32),
            jax.ShapeDtypeStruct((1, 1), jnp.float32),
        ],
        scratch_shapes=[
            pltpu.VMEM((1, _N_E), jnp.float32),
            pltpu.SMEM((1, 1), jnp.float32),
        ],
    )(z, weight, w2, zsq, wsq, col_iota, ones_row)

    z_q = zq
    loss = loss[0, 0]
    perplexity = perp[0, 0]
    return (z_q, loss, (perplexity, oh, idx))


# speedup vs baseline: 2.3410x; 2.3410x over previous
import functools

import jax
import jax.numpy as jnp
from jax.experimental import pallas as pl
from jax.experimental.pallas import tpu as pltpu

_N_E = 8192
_E_DIM = 32
_BETA = 0.25
_N_TOK = 8192
_BLK = 256
_NBLK = _N_TOK // _BLK


def _vq_body(z_ref, w_ref, w2_ref, zsq_ref, wsq_ref, iota_ref, ones_ref,
             oh_ref, zq_ref, idx_ref, loss_ref, perp_ref,
             counts_ref, lsum_ref):
    i = pl.program_id(0)
    zb = z_ref[...]
    w = w_ref[...]
    zsq = zsq_ref[...]
    wsq = wsq_ref[...]

    mm2 = jax.lax.dot_general(
        zb, w2_ref[...], dimension_numbers=(((1,), (1,)), ((), ())),
        preferred_element_type=jnp.float32)
    d = zsq + wsq - mm2

    half = _N_E // 2
    iota = iota_ref[...]
    d_a, d_b = d[:, :half], d[:, half:]
    i_a, i_b = iota[:, :half], iota[:, half:]
    v_a = jnp.min(d_a, axis=1, keepdims=True)
    j_a = jnp.min(jnp.where(d_a == v_a, i_a, _N_E), axis=1, keepdims=True)
    v_b = jnp.min(d_b, axis=1, keepdims=True)
    j_b = jnp.min(jnp.where(d_b == v_b, i_b, _N_E), axis=1, keepdims=True)
    v_a16 = v_a.astype(jnp.bfloat16).astype(jnp.float32)
    keep_a = (v_a16 < v_b) | ((v_a16 == v_b) & (j_a < j_b))
    idx = jnp.where(keep_a, j_a, j_b)

    oh = jnp.where(iota == idx, 1.0, 0.0).astype(jnp.float32)
    oh_ref[...] = oh
    idx_ref[...] = idx

    zq = jax.lax.dot_general(
        oh, w, dimension_numbers=(((1,), (0,)), ((), ())),
        preferred_element_type=jnp.float32)
    zq_ref[...] = zq

    diff = zq - zb
    part_loss = jnp.sum(diff * diff)
    part_counts = jax.lax.dot_general(
        ones_ref[...], oh, dimension_numbers=(((1,), (0,)), ((), ())),
        preferred_element_type=jnp.float32)

    @pl.when(i == 0)
    def _init():
        lsum_ref[0, 0] = part_loss
        counts_ref[...] = part_counts

    @pl.when(i > 0)
    def _acc():
        lsum_ref[0, 0] += part_loss
        counts_ref[...] += part_counts

    @pl.when(i == _NBLK - 1)
    def _fini():
        mse = lsum_ref[0, 0] / (_N_TOK * _E_DIM)
        loss_ref[...] = jnp.reshape(_BETA * mse + mse, (1, 1))
        e_mean = counts_ref[...] / _N_TOK
        ent = jnp.sum(e_mean * jnp.log(e_mean + 1e-10), axis=1, keepdims=True)
        perp_ref[...] = jnp.exp(-ent)


@functools.partial(jax.jit, static_argnames=())
def kernel(z, weight):
    zsq = jnp.sum(z ** 2, axis=1, keepdims=True)
    wsq = jnp.sum(weight ** 2, axis=1)[None, :]
    w2 = weight * 2.0
    col_iota = jax.lax.broadcasted_iota(jnp.int32, (1, _N_E), 1)
    ones_row = jnp.ones((1, _BLK), jnp.float32)

    oh, zq, idx, loss, perp = pl.pallas_call(
        _vq_body,
        grid=(_NBLK,),
        in_specs=[
            pl.BlockSpec((_BLK, _E_DIM), lambda i: (i, 0)),
            pl.BlockSpec((_N_E, _E_DIM), lambda i: (0, 0)),
            pl.BlockSpec((_N_E, _E_DIM), lambda i: (0, 0)),
            pl.BlockSpec((_BLK, 1), lambda i: (i, 0)),
            pl.BlockSpec((1, _N_E), lambda i: (0, 0)),
            pl.BlockSpec((1, _N_E), lambda i: (0, 0)),
            pl.BlockSpec((1, _BLK), lambda i: (0, 0)),
        ],
        out_specs=[
            pl.BlockSpec((_BLK, _N_E), lambda i: (i, 0)),
            pl.BlockSpec((_BLK, _E_DIM), lambda i: (i, 0)),
            pl.BlockSpec((_BLK, 1), lambda i: (i, 0)),
            pl.BlockSpec((1, 1), lambda i: (0, 0)),
            pl.BlockSpec((1, 1), lambda i: (0, 0)),
        ],
        out_shape=[
            jax.ShapeDtypeStruct((_N_TOK, _N_E), jnp.float32),
            jax.ShapeDtypeStruct((_N_TOK, _E_DIM), jnp.float32),
            jax.ShapeDtypeStruct((_N_TOK, 1), jnp.int32),
            jax.ShapeDtypeStruct((1, 1), jnp.float32),
            jax.ShapeDtypeStruct((1, 1), jnp.float32),
        ],
        scratch_shapes=[
            pltpu.VMEM((1, _N_E), jnp.float32),
            pltpu.SMEM((1, 1), jnp.float32),
        ],
    )(z, weight, w2, zsq, wsq, col_iota, ones_row)

    z_q = zq
    loss = loss[0, 0]
    perplexity = perp[0, 0]
    return (z_q, loss, (perplexity, oh, idx))


# speedup vs baseline: 2.4156x; 1.0319x over previous
import functools

import jax
import jax.numpy as jnp
from jax.experimental import pallas as pl
from jax.experimental.pallas import tpu as pltpu

_N_E = 8192
_E_DIM = 32
_BETA = 0.25
_N_TOK = 8192
_BLK = 512
_NBLK = _N_TOK // _BLK


def _vq_body(z_ref, w_ref, w2_ref, zsq_ref, wsq_ref, iota_ref, ones_ref,
             oh_ref, zq_ref, idx_ref, loss_ref, perp_ref,
             counts_ref, lsum_ref):
    i = pl.program_id(0)
    zb = z_ref[...]
    w = w_ref[...]
    zsq = zsq_ref[...]
    wsq = wsq_ref[...]

    mm2 = jax.lax.dot_general(
        zb, w2_ref[...], dimension_numbers=(((1,), (1,)), ((), ())),
        preferred_element_type=jnp.float32)
    d = zsq + wsq - mm2

    half = _N_E // 2
    iota = iota_ref[...]
    d_a, d_b = d[:, :half], d[:, half:]
    i_a, i_b = iota[:, :half], iota[:, half:]
    v_a = jnp.min(d_a, axis=1, keepdims=True)
    j_a = jnp.min(jnp.where(d_a == v_a, i_a, _N_E), axis=1, keepdims=True)
    v_b = jnp.min(d_b, axis=1, keepdims=True)
    j_b = jnp.min(jnp.where(d_b == v_b, i_b, _N_E), axis=1, keepdims=True)
    v_a16 = v_a.astype(jnp.bfloat16).astype(jnp.float32)
    keep_a = (v_a16 < v_b) | ((v_a16 == v_b) & (j_a < j_b))
    idx = jnp.where(keep_a, j_a, j_b)

    oh = jnp.where(iota == idx, 1.0, 0.0).astype(jnp.float32)
    oh_ref[...] = oh
    idx_ref[...] = idx

    zq = jax.lax.dot_general(
        oh, w, dimension_numbers=(((1,), (0,)), ((), ())),
        preferred_element_type=jnp.float32)
    zq_ref[...] = zq

    diff = zq - zb
    part_loss = jnp.sum(diff * diff)
    part_counts = jax.lax.dot_general(
        ones_ref[...], oh, dimension_numbers=(((1,), (0,)), ((), ())),
        preferred_element_type=jnp.float32)

    @pl.when(i == 0)
    def _init():
        lsum_ref[0, 0] = part_loss
        counts_ref[...] = part_counts

    @pl.when(i > 0)
    def _acc():
        lsum_ref[0, 0] += part_loss
        counts_ref[...] += part_counts

    @pl.when(i == _NBLK - 1)
    def _fini():
        mse = lsum_ref[0, 0] / (_N_TOK * _E_DIM)
        loss_ref[...] = jnp.reshape(_BETA * mse + mse, (1, 1))
        e_mean = counts_ref[...] / _N_TOK
        ent = jnp.sum(e_mean * jnp.log(e_mean + 1e-10), axis=1, keepdims=True)
        perp_ref[...] = jnp.exp(-ent)


@functools.partial(jax.jit, static_argnames=())
def kernel(z, weight):
    zsq = jnp.sum(z ** 2, axis=1, keepdims=True)
    wsq = jnp.sum(weight ** 2, axis=1)[None, :]
    w2 = weight * 2.0
    col_iota = jax.lax.broadcasted_iota(jnp.int32, (1, _N_E), 1)
    ones_row = jnp.ones((1, _BLK), jnp.float32)

    oh, zq, idx, loss, perp = pl.pallas_call(
        _vq_body,
        grid=(_NBLK,),
        in_specs=[
            pl.BlockSpec((_BLK, _E_DIM), lambda i: (i, 0)),
            pl.BlockSpec((_N_E, _E_DIM), lambda i: (0, 0)),
            pl.BlockSpec((_N_E, _E_DIM), lambda i: (0, 0)),
            pl.BlockSpec((_BLK, 1), lambda i: (i, 0)),
            pl.BlockSpec((1, _N_E), lambda i: (0, 0)),
            pl.BlockSpec((1, _N_E), lambda i: (0, 0)),
            pl.BlockSpec((1, _BLK), lambda i: (0, 0)),
        ],
        out_specs=[
            pl.BlockSpec((_BLK, _N_E), lambda i: (i, 0)),
            pl.BlockSpec((_BLK, _E_DIM), lambda i: (i, 0)),
            pl.BlockSpec((_BLK, 1), lambda i: (i, 0)),
            pl.BlockSpec((1, 1), lambda i: (0, 0)),
            pl.BlockSpec((1, 1), lambda i: (0, 0)),
        ],
        out_shape=[
            jax.ShapeDtypeStruct((_N_TOK, _N_E), jnp.float32),
            jax.ShapeDtypeStruct((_N_TOK, _E_DIM), jnp.float32),
            jax.ShapeDtypeStruct((_N_TOK, 1), jnp.int32),
            jax.ShapeDtypeStruct((1, 1), jnp.float32),
            jax.ShapeDtypeStruct((1, 1), jnp.float32),
        ],
        scratch_shapes=[
            pltpu.VMEM((1, _N_E), jnp.float32),
            pltpu.SMEM((1, 1), jnp.float32),
        ],
    )(z, weight, w2, zsq, wsq, col_iota, ones_row)

    z_q = zq
    loss = loss[0, 0]
    perplexity = perp[0, 0]
    return (z_q, loss, (perplexity, oh, idx))


# speedup vs baseline: 2.9378x; 1.2162x over previous
import functools

import jax
import jax.numpy as jnp
from jax.experimental import pallas as pl
from jax.experimental.pallas import tpu as pltpu
import jax.experimental.pallas.tpu_sc as plsc

_N_E = 8192
_E_DIM = 32
_BETA = 0.25
_N_TOK = 8192
_BLK = 512
_NBLK = _N_TOK // _BLK
_PAD = 128
_SUBS = 32
_SC_TOK = _N_TOK // _SUBS


def _vq_body(z_ref, w2_ref, zsq_ref, wsq_ref, iota_ref, ones_ref,
             oh_ref, idx_ref, loss_ref, perp_ref,
             counts_ref, lsum_ref):
    i = pl.program_id(0)
    zb = z_ref[...]
    zsq = zsq_ref[...]
    wsq = wsq_ref[...]

    mm2 = jax.lax.dot_general(
        zb, w2_ref[...], dimension_numbers=(((1,), (1,)), ((), ())),
        preferred_element_type=jnp.float32)
    d = zsq + wsq - mm2

    half = _N_E // 2
    iota = iota_ref[...]
    d_a, d_b = d[:, :half], d[:, half:]
    i_a, i_b = iota[:, :half], iota[:, half:]
    v_a = jnp.min(d_a, axis=1, keepdims=True)
    j_a = jnp.min(jnp.where(d_a == v_a, i_a, _N_E), axis=1, keepdims=True)
    v_b = jnp.min(d_b, axis=1, keepdims=True)
    j_b = jnp.min(jnp.where(d_b == v_b, i_b, _N_E), axis=1, keepdims=True)
    v_a16 = v_a.astype(jnp.bfloat16).astype(jnp.float32)
    keep_a = (v_a16 < v_b) | ((v_a16 == v_b) & (j_a < j_b))
    idx = jnp.where(keep_a, j_a, j_b)

    oh = jnp.where(iota == idx, 1.0, 0.0).astype(jnp.float32)
    oh_ref[...] = oh
    idx_ref[...] = idx

    part_loss = jnp.sum(jnp.where(keep_a, v_a, v_b))
    part_counts = jax.lax.dot_general(
        ones_ref[...], oh, dimension_numbers=(((1,), (0,)), ((), ())),
        preferred_element_type=jnp.float32)

    @pl.when(i == 0)
    def _init():
        lsum_ref[0, 0] = part_loss
        counts_ref[...] = part_counts

    @pl.when(i > 0)
    def _acc():
        lsum_ref[0, 0] += part_loss
        counts_ref[...] += part_counts

    @pl.when(i == _NBLK - 1)
    def _fini():
        mse = lsum_ref[0, 0] / (_N_TOK * _E_DIM)
        loss_ref[...] = jnp.reshape(_BETA * mse + mse, (1, 1))
        e_mean = counts_ref[...] / _N_TOK
        ent = jnp.sum(e_mean * jnp.log(e_mean + 1e-10), axis=1, keepdims=True)
        perp_ref[...] = jnp.exp(-ent)


def _sc_gather_body(idx_hbm, w_hbm, zq_hbm):
    c = jax.lax.axis_index("c")
    s = jax.lax.axis_index("s")
    start = (c * 16 + s) * _SC_TOK

    def scoped(idx_vmem, rows_vmem, sem_i, sem_g, sem_o):
        cp = pltpu.make_async_copy(
            idx_hbm.at[pl.ds(start, _SC_TOK)], idx_vmem, sem_i)
        cp.start()
        cp.wait()
        gp = pltpu.make_async_copy(w_hbm.at[idx_vmem], rows_vmem, sem_g)
        gp.start()
        gp.wait()
        op = pltpu.make_async_copy(
            rows_vmem, zq_hbm.at[pl.ds(start, _SC_TOK)], sem_o)
        op.start()
        op.wait()

    pl.run_scoped(
        scoped,
        idx_vmem=pltpu.VMEM((_SC_TOK,), jnp.int32),
        rows_vmem=pltpu.VMEM((_SC_TOK, _PAD), jnp.float32),
        sem_i=pltpu.SemaphoreType.DMA,
        sem_g=pltpu.SemaphoreType.DMA,
        sem_o=pltpu.SemaphoreType.DMA,
    )


def _sc_gather(idx_flat, w_pad):
    return pl.kernel(
        _sc_gather_body,
        out_type=jax.ShapeDtypeStruct((_N_TOK, _PAD), jnp.float32),
        mesh=plsc.VectorSubcoreMesh(core_axis_name="c", subcore_axis_name="s"),
    )(idx_flat, w_pad)


@functools.partial(jax.jit, static_argnames=())
def kernel(z, weight):
    zsq = jnp.sum(z ** 2, axis=1, keepdims=True)
    wsq = jnp.sum(weight ** 2, axis=1)[None, :]
    w2 = weight * 2.0
    col_iota = jax.lax.broadcasted_iota(jnp.int32, (1, _N_E), 1)
    ones_row = jnp.ones((1, _BLK), jnp.float32)
    w_pad = jnp.pad(weight, ((0, 0), (0, _PAD - _E_DIM)))

    oh, idx, loss, perp = pl.pallas_call(
        _vq_body,
        grid=(_NBLK,),
        in_specs=[
            pl.BlockSpec((_BLK, _E_DIM), lambda i: (i, 0)),
            pl.BlockSpec((_N_E, _E_DIM), lambda i: (0, 0)),
            pl.BlockSpec((_BLK, 1), lambda i: (i, 0)),
            pl.BlockSpec((1, _N_E), lambda i: (0, 0)),
            pl.BlockSpec((1, _N_E), lambda i: (0, 0)),
            pl.BlockSpec((1, _BLK), lambda i: (0, 0)),
        ],
        out_specs=[
            pl.BlockSpec((_BLK, _N_E), lambda i: (i, 0)),
            pl.BlockSpec((_BLK, 1), lambda i: (i, 0)),
            pl.BlockSpec((1, 1), lambda i: (0, 0)),
            pl.BlockSpec((1, 1), lambda i: (0, 0)),
        ],
        out_shape=[
            jax.ShapeDtypeStruct((_N_TOK, _N_E), jnp.float32),
            jax.ShapeDtypeStruct((_N_TOK, 1), jnp.int32),
            jax.ShapeDtypeStruct((1, 1), jnp.float32),
            jax.ShapeDtypeStruct((1, 1), jnp.float32),
        ],
        scratch_shapes=[
            pltpu.VMEM((1, _N_E), jnp.float32),
            pltpu.SMEM((1, 1), jnp.float32),
        ],
    )(z, w2, zsq, wsq, col_iota, ones_row)

    zq_pad = _sc_gather(idx[:, 0], w_pad)
    z_q = zq_pad[:, :_E_DIM]
    loss = loss[0, 0]
    perplexity = perp[0, 0]
    return (z_q, loss, (perplexity, oh, idx))
